# baseline (speedup 1.0000x reference)
import jax
import jax.numpy as jnp
from jax import lax
from jax.experimental import pallas as pl
from jax.experimental.pallas import tpu as pltpu
from jax.experimental.pallas import tpu_sc as plsc

T = 2048
D = 1024
DFF = 768
E = 8
K = 2
BM = 128
NB = (T * K + E * BM) // BM
NPAD = NB * BM
NC, NS = 2, 16
NW = NC * NS
TPW = T // NW
CH = 32
LANES = 16

def _mesh():
    return plsc.VectorSubcoreMesh(core_axis_name="c", subcore_axis_name="s",
                                  num_cores=NC, num_subcores=NS)


def _dispatch_body(x_hbm, pe_hbm, po_hbm, xs_hbm, xbuf, idxe, idxo):
    wid = lax.axis_index("s") * NC + lax.axis_index("c")
    tb = wid * TPW
    pltpu.sync_copy(x_hbm.at[pl.ds(tb, TPW)], xbuf)
    pltpu.sync_copy(pe_hbm.at[pl.ds(tb, TPW)], idxe)
    pltpu.sync_copy(po_hbm.at[pl.ds(tb, TPW)], idxo)
    pltpu.sync_copy(xbuf, xs_hbm.at[idxe])
    pltpu.sync_copy(xbuf, xs_hbm.at[idxo])


def _dispatch(hidden_states, pos_e, pos_o):
    return pl.kernel(
        _dispatch_body,
        out_type=jax.ShapeDtypeStruct((NPAD, D), jnp.float32),
        mesh=_mesh(),
        scratch_types=[
            pltpu.VMEM((TPW, D), jnp.float32),
            pltpu.VMEM((TPW,), jnp.int32),
            pltpu.VMEM((TPW,), jnp.int32),
        ],
    )(hidden_states, pos_e, pos_o)


def _gather_body(y_hbm, pe_hbm, po_hbm, y0_hbm, y1_hbm, idxe, idxo, b0, b1):
    wid = lax.axis_index("s") * NC + lax.axis_index("c")
    tb = wid * TPW
    for c in range(TPW // CH):
        pltpu.sync_copy(pe_hbm.at[pl.ds(tb + c * CH, CH)], idxe)
        pltpu.sync_copy(po_hbm.at[pl.ds(tb + c * CH, CH)], idxo)
        pltpu.sync_copy(y_hbm.at[idxe], b0)
        pltpu.sync_copy(y_hbm.at[idxo], b1)
        pltpu.sync_copy(b0, y0_hbm.at[pl.ds(tb + c * CH, CH)])
        pltpu.sync_copy(b1, y1_hbm.at[pl.ds(tb + c * CH, CH)])


def _gather(y_sorted, pos_e, pos_o):
    return pl.kernel(
        _gather_body,
        out_type=(jax.ShapeDtypeStruct((T, D), jnp.float32),
                  jax.ShapeDtypeStruct((T, D), jnp.float32)),
        mesh=_mesh(),
        scratch_types=[
            pltpu.VMEM((CH,), jnp.int32),
            pltpu.VMEM((CH,), jnp.int32),
            pltpu.VMEM((CH, D), jnp.float32),
            pltpu.VMEM((CH, D), jnp.float32),
        ],
    )(y_sorted, pos_e, pos_o)


BT = 256


def _wadd_body(y0_ref, y1_ref, w_ref, o_ref):
    o_ref[...] = (y0_ref[...] * w_ref[:, :1] + y1_ref[...] * w_ref[:, 1:2])


def _wadd(y0, y1, topk_weights):
    return pl.pallas_call(
        _wadd_body,
        grid=(T // BT,),
        in_specs=[
            pl.BlockSpec((BT, D), lambda i: (i, 0)),
            pl.BlockSpec((BT, D), lambda i: (i, 0)),
            pl.BlockSpec((BT, K), lambda i: (i, 0)),
        ],
        out_specs=pl.BlockSpec((BT, D), lambda i: (i, 0)),
        out_shape=jax.ShapeDtypeStruct((T, D), jnp.float32),
    )(y0, y1, topk_weights)


def _gemm_body(be_ref, x_ref, w1_ref, w2_ref, o_ref):
    x = x_ref[...]
    gu = lax.dot_general(x, w1_ref[0], (((1,), (1,)), ((), ())),
                         preferred_element_type=jnp.float32)
    gate = gu[:, :DFF]
    up = gu[:, DFF:]
    act = gate * lax.logistic(gate) * up
    y = lax.dot_general(act, w2_ref[0], (((1,), (1,)), ((), ())),
                        preferred_element_type=jnp.float32)
    o_ref[...] = y


def _gemm(block_expert, x_sorted, w1, w2):
    grid_spec = pltpu.PrefetchScalarGridSpec(
        num_scalar_prefetch=1,
        grid=(NB,),
        in_specs=[
            pl.BlockSpec((BM, D), lambda i, be: (i, 0)),
            pl.BlockSpec((1, 2 * DFF, D), lambda i, be: (be[i], 0, 0)),
            pl.BlockSpec((1, D, DFF), lambda i, be: (be[i], 0, 0)),
        ],
        out_specs=pl.BlockSpec((BM, D), lambda i, be: (i, 0)),
    )
    return pl.pallas_call(
        _gemm_body,
        grid_spec=grid_spec,
        out_shape=jax.ShapeDtypeStruct((NPAD, D), jnp.float32),
    )(block_expert, x_sorted, w1, w2)


def kernel(hidden_states, topk_weights, topk_ids, w1, w2):
    ids = topk_ids.astype(jnp.int32)
    flat_e = ids.reshape(-1)
    oh = (flat_e[:, None] == jnp.arange(E, dtype=jnp.int32)[None, :]).astype(jnp.int32)
    incl = jnp.cumsum(oh, axis=0)
    counts = incl[-1]
    rank = jnp.take_along_axis(incl, flat_e[:, None], axis=1)[:, 0] - 1
    padded = ((counts + BM - 1) // BM) * BM
    offs = jnp.cumsum(padded) - padded
    pos = (offs[flat_e] + rank).reshape(T, K)
    pos_e = pos[:, 0]
    pos_o = pos[:, 1]
    cum = jnp.cumsum(padded)
    block_expert = jnp.minimum(
        jnp.searchsorted(cum, jnp.arange(NB, dtype=jnp.int32) * BM, side="right"),
        E - 1).astype(jnp.int32)
    x_sorted = _dispatch(hidden_states, pos_e, pos_o)
    y_sorted = _gemm(block_expert, x_sorted, w1, w2)
    y0, y1 = _gather(y_sorted, pos_e, pos_o)
    return _wadd(y0, y1, topk_weights)


# speedup vs baseline: 1.4524x; 1.4524x over previous
import jax
import jax.numpy as jnp
from jax import lax
from jax.experimental import pallas as pl
from jax.experimental.pallas import tpu as pltpu
from jax.experimental.pallas import tpu_sc as plsc

T = 2048
D = 1024
DFF = 768
E = 8
K = 2
BM = 256
NB = (T * K + E * BM) // BM
NPAD = NB * BM
NC, NS = 2, 16
NW = NC * NS
TPW = T // NW
CH = 32
LANES = 16

def _cumsum_lanes(x, n):
    s = 1
    while s < n:
        shifted = jnp.concatenate(
            [jnp.zeros((x.shape[0], s), x.dtype), x[:, :-s]], axis=1)
        x = x + shifted
        s *= 2
    return x


def _meta_body(ids_ref, pe_ref, po_ref, be_ref):
    ids = ids_ref[...]
    e0 = ids[:, 0][None, :]
    e1 = ids[:, 1][None, :]
    lanes_e = jax.lax.broadcasted_iota(jnp.int32, (E, T), 0)
    oh0 = (lanes_e == e0)
    oh1 = (lanes_e == e1)
    rowcnt = oh0.astype(jnp.int32) + oh1.astype(jnp.int32)
    incl = _cumsum_lanes(rowcnt, T)
    excl = incl - rowcnt
    rank0 = jnp.sum(jnp.where(oh0, excl, 0), axis=0)
    rank1 = jnp.sum(jnp.where(oh1, excl, 0), axis=0) + (ids[:, 0] == ids[:, 1])
    counts = incl[:, T - 1:T]
    padded = ((counts + BM - 1) // BM) * BM
    cum = padded
    s = 1
    while s < E:
        cum = cum + jnp.concatenate(
            [jnp.zeros((s, 1), jnp.int32), cum[:-s, :]], axis=0)
        s *= 2
    offs = cum - padded
    off0 = jnp.sum(jnp.where(oh0, offs, 0), axis=0)
    off1 = jnp.sum(jnp.where(oh1, offs, 0), axis=0)
    pe_ref[...] = (off0 + rank0)[None, :]
    po_ref[...] = (off1 + rank1)[None, :]
    bstart = jax.lax.broadcasted_iota(jnp.int32, (E, NB), 1) * BM
    be = jnp.sum((cum <= bstart).astype(jnp.int32), axis=0)
    be_ref[...] = jnp.minimum(be, E - 1)[None, :]


def _meta(ids):
    pe, po, be = pl.pallas_call(
        _meta_body,
        out_shape=(jax.ShapeDtypeStruct((1, T), jnp.int32),
                   jax.ShapeDtypeStruct((1, T), jnp.int32),
                   jax.ShapeDtypeStruct((1, NB), jnp.int32)),
    )(ids)
    return pe.reshape(T), po.reshape(T), be.reshape(NB)


def _mesh():
    return plsc.VectorSubcoreMesh(core_axis_name="c", subcore_axis_name="s",
                                  num_cores=NC, num_subcores=NS)


def _dispatch_body(x_hbm, pe_hbm, po_hbm, xs_hbm, xbuf, idxe, idxo):
    wid = lax.axis_index("s") * NC + lax.axis_index("c")
    tb = wid * TPW
    pltpu.sync_copy(x_hbm.at[pl.ds(tb, TPW)], xbuf)
    pltpu.sync_copy(pe_hbm.at[pl.ds(tb, TPW)], idxe)
    pltpu.sync_copy(po_hbm.at[pl.ds(tb, TPW)], idxo)
    pltpu.sync_copy(xbuf, xs_hbm.at[idxe])
    pltpu.sync_copy(xbuf, xs_hbm.at[idxo])


def _dispatch(hidden_states, pos_e, pos_o):
    return pl.kernel(
        _dispatch_body,
        out_type=jax.ShapeDtypeStruct((NPAD, D), jnp.float32),
        mesh=_mesh(),
        scratch_types=[
            pltpu.VMEM((TPW, D), jnp.float32),
            pltpu.VMEM((TPW,), jnp.int32),
            pltpu.VMEM((TPW,), jnp.int32),
        ],
    )(hidden_states, pos_e, pos_o)


def _gather_body(y_hbm, pe_hbm, po_hbm, y0_hbm, y1_hbm, idxe, idxo, b0, b1):
    wid = lax.axis_index("s") * NC + lax.axis_index("c")
    tb = wid * TPW
    for c in range(TPW // CH):
        pltpu.sync_copy(pe_hbm.at[pl.ds(tb + c * CH, CH)], idxe)
        pltpu.sync_copy(po_hbm.at[pl.ds(tb + c * CH, CH)], idxo)
        pltpu.sync_copy(y_hbm.at[idxe], b0)
        pltpu.sync_copy(y_hbm.at[idxo], b1)
        pltpu.sync_copy(b0, y0_hbm.at[pl.ds(tb + c * CH, CH)])
        pltpu.sync_copy(b1, y1_hbm.at[pl.ds(tb + c * CH, CH)])


def _gather(y_sorted, pos_e, pos_o):
    return pl.kernel(
        _gather_body,
        out_type=(jax.ShapeDtypeStruct((T, D), jnp.float32),
                  jax.ShapeDtypeStruct((T, D), jnp.float32)),
        mesh=_mesh(),
        scratch_types=[
            pltpu.VMEM((CH,), jnp.int32),
            pltpu.VMEM((CH,), jnp.int32),
            pltpu.VMEM((CH, D), jnp.float32),
            pltpu.VMEM((CH, D), jnp.float32),
        ],
    )(y_sorted, pos_e, pos_o)


BT = 256


def _wadd_body(y0_ref, y1_ref, w_ref, o_ref):
    o_ref[...] = (y0_ref[...] * w_ref[:, :1] + y1_ref[...] * w_ref[:, 1:2])


def _wadd(y0, y1, topk_weights):
    return pl.pallas_call(
        _wadd_body,
        grid=(T // BT,),
        in_specs=[
            pl.BlockSpec((BT, D), lambda i: (i, 0)),
            pl.BlockSpec((BT, D), lambda i: (i, 0)),
            pl.BlockSpec((BT, K), lambda i: (i, 0)),
        ],
        out_specs=pl.BlockSpec((BT, D), lambda i: (i, 0)),
        out_shape=jax.ShapeDtypeStruct((T, D), jnp.float32),
    )(y0, y1, topk_weights)


def _gemm_body(be_ref, x_ref, w1_ref, w2_ref, o_ref):
    x = x_ref[...]
    gu = lax.dot_general(x, w1_ref[0], (((1,), (1,)), ((), ())),
                         preferred_element_type=jnp.float32)
    gate = gu[:, :DFF]
    up = gu[:, DFF:]
    act = gate * lax.logistic(gate) * up
    y = lax.dot_general(act, w2_ref[0], (((1,), (1,)), ((), ())),
                        preferred_element_type=jnp.float32)
    o_ref[...] = y


def _gemm(block_expert, x_sorted, w1, w2):
    grid_spec = pltpu.PrefetchScalarGridSpec(
        num_scalar_prefetch=1,
        grid=(NB,),
        in_specs=[
            pl.BlockSpec((BM, D), lambda i, be: (i, 0)),
            pl.BlockSpec((1, 2 * DFF, D), lambda i, be: (be[i], 0, 0)),
            pl.BlockSpec((1, D, DFF), lambda i, be: (be[i], 0, 0)),
        ],
        out_specs=pl.BlockSpec((BM, D), lambda i, be: (i, 0)),
    )
    return pl.pallas_call(
        _gemm_body,
        grid_spec=grid_spec,
        out_shape=jax.ShapeDtypeStruct((NPAD, D), jnp.float32),
    )(block_expert, x_sorted, w1, w2)


def kernel(hidden_states, topk_weights, topk_ids, w1, w2):
    ids = topk_ids.astype(jnp.int32)
    pos_e, pos_o, block_expert = _meta(ids)
    x_sorted = _dispatch(hidden_states, pos_e, pos_o)
    y_sorted = _gemm(block_expert, x_sorted, w1, w2)
    y0, y1 = _gather(y_sorted, pos_e, pos_o)
    return _wadd(y0, y1, topk_weights)
